# baseline (speedup 1.0000x reference)
import jax
import jax.numpy as jnp
from jax.experimental import pallas as pl
from jax.experimental.pallas import tpu as pltpu
from jax.experimental.pallas import tpu_sc as plsc

DIM = 2048
B = 2048
E = 4
H = 4
DH = DIM // H
BT = 512
RB = B // BT
NQ = 4
QW = DIM // NQ
N1 = 16
HB = (4 * DIM) // N1
BTM = 1024
RBM = B // BTM

F32 = jnp.float32
BF16 = jnp.bfloat16


def _prep_kernel(q_ref, k_ref, v_ref, wp_ref, wq_ref, wkv_ref, wpb_ref):
    e = pl.program_id(1)
    wq_ref[...] = q_ref[...].astype(BF16)
    wkv_ref[...] = (k_ref[...] + v_ref[...]).astype(BF16)

    @pl.when(e == 0)
    def _():
        wp = wp_ref[...].reshape(DH, H, QW)
        wpb_ref[...] = (
            wp.transpose(1, 0, 2).reshape(DIM, QW).astype(BF16))


def _sc_top2_mask(p):
    mesh = plsc.ScalarSubcoreMesh(axis_name="core", num_cores=2)
    half = B // 2
    hw = half * E

    @pl.kernel(out_type=jax.ShapeDtypeStruct((B * E,), F32), mesh=mesh,
               scratch_types=[pltpu.SMEM((hw,), F32),
                              pltpu.SMEM((hw,), F32),
                              pltpu.SemaphoreType.DMA])
    def sc_mask(p_hbm, m_hbm, pin, mout, sem):
        c = jax.lax.axis_index("core")
        pltpu.async_copy(p_hbm.at[pl.ds(c * hw, hw)], pin, sem).wait()

        @pl.loop(0, half)
        def _(t):
            p0 = pin[4 * t + 0]
            p1 = pin[4 * t + 1]
            p2 = pin[4 * t + 2]
            p3 = pin[4 * t + 3]
            b01 = p1 > p0
            m01 = jnp.where(b01, p1, p0)
            i01 = jnp.where(b01, 1, 0)
            b23 = p3 > p2
            m23 = jnp.where(b23, p3, p2)
            i23 = jnp.where(b23, 3, 2)
            i1 = jnp.where(m23 > m01, i23, i01)
            q0 = jnp.where(i1 == 0, -1.0, p0)
            q1 = jnp.where(i1 == 1, -1.0, p1)
            q2 = jnp.where(i1 == 2, -1.0, p2)
            q3 = jnp.where(i1 == 3, -1.0, p3)
            c01 = q1 > q0
            n01 = jnp.where(c01, q1, q0)
            j01 = jnp.where(c01, 1, 0)
            c23 = q3 > q2
            n23 = jnp.where(c23, q3, q2)
            j23 = jnp.where(c23, 3, 2)
            i2 = jnp.where(n23 > n01, j23, j01)
            mout[4 * t + 0] = jnp.where((i1 == 0) | (i2 == 0), 1.0, 0.0)
            mout[4 * t + 1] = jnp.where((i1 == 1) | (i2 == 1), 1.0, 0.0)
            mout[4 * t + 2] = jnp.where((i1 == 2) | (i2 == 2), 1.0, 0.0)
            mout[4 * t + 3] = jnp.where((i1 == 3) | (i2 == 3), 1.0, 0.0)

        pltpu.async_copy(mout, m_hbm.at[pl.ds(c * hw, hw)], sem).wait()

    return sc_mask(p.reshape(B * E)).reshape(B, E)


def _pre_kernel(x_ref, y_ref, g1_ref, b1_ref, wg_ref, bg_ref,
                yn_ref, xb_ref, w_ref):
    xv = x_ref[...]
    yv = y_ref[...]
    m = jnp.mean(yv, axis=1, keepdims=True)
    v = jnp.mean((yv - m) ** 2, axis=1, keepdims=True)
    yn = (yv - m) / jnp.sqrt(v + 1e-5) * g1_ref[...] + b1_ref[...]
    yn_ref[...] = yn.astype(BF16)
    xb_ref[...] = xv.astype(BF16)
    logits = jnp.dot(xv.astype(BF16), wg_ref[...].astype(BF16),
                     preferred_element_type=F32) + bg_ref[...]
    m1 = jnp.max(logits, axis=1, keepdims=True)
    p = jnp.exp(logits - m1)
    w_ref[...] = p / jnp.sum(p, axis=1, keepdims=True)


def _qkv_kernel(xb_ref, yn_ref, wq_ref, wkv_ref, w_ref, mask_ref,
                attn_ref, acc_ref):
    e = pl.program_id(1)
    contrib = (jnp.dot(yn_ref[...], wq_ref[0], preferred_element_type=F32)
               + jnp.dot(xb_ref[...], wkv_ref[0], preferred_element_type=F32))
    lane = jax.lax.broadcasted_iota(jnp.int32, (BT, E), 1)
    wcol = jnp.sum(jnp.where(lane == e, w_ref[...] * mask_ref[...], 0.0),
                   axis=1, keepdims=True)
    contrib = contrib * wcol

    @pl.when(e == 0)
    def _():
        acc_ref[...] = contrib

    @pl.when(e > 0)
    def _():
        acc_ref[...] += contrib

    @pl.when(e == E - 1)
    def _():
        acc = acc_ref[...]
        scale = DH ** -0.5
        heads = [acc[:, h * DH:(h + 1) * DH] for h in range(H)]
        s = [[None] * H for _ in range(H)]
        for h in range(H):
            for g in range(h, H):
                s[h][g] = jnp.sum(heads[h] * heads[g], axis=1,
                                  keepdims=True) * scale
                s[g][h] = s[h][g]
        outs = []
        for h in range(H):
            mx = s[h][0]
            for g in range(1, H):
                mx = jnp.maximum(mx, s[h][g])
            es = [jnp.exp(s[h][g] - mx) for g in range(H)]
            den = es[0] + es[1] + es[2] + es[3]
            o = (es[0] / den) * heads[0]
            for g in range(1, H):
                o += (es[g] / den) * heads[g]
            outs.append(o)
        attn_ref[...] = jnp.concatenate(outs, axis=1).astype(BF16)


def _proj_kernel(y_ref, attn_ref, wp_ref, bp_ref, g2_ref, be2_ref, b2_ref,
                 out1_ref, h2_ref):
    yv = y_ref[...]
    m = jnp.mean(yv, axis=1, keepdims=True)
    v = jnp.mean((yv - m) ** 2, axis=1, keepdims=True)
    h2 = (yv - m) / jnp.sqrt(v + 1e-5) * g2_ref[...] + be2_ref[...]
    h2_ref[...] = h2.astype(BF16)
    out1_ref[...] = (yv
                     + jnp.dot(attn_ref[...], wp_ref[...],
                               preferred_element_type=F32)
                     + bp_ref[...] + b2_ref[...]).astype(BF16)


def _mlp_kernel(out1_ref, h2_ref, w1_ref, b1_ref, w2_ref, out_ref):
    n = pl.program_id(1)

    @pl.when(n == 0)
    def _():
        out_ref[...] = out1_ref[...].astype(F32)

    h2v = h2_ref[...]
    st = HB // 2
    acc = None
    for t in range(2):
        w1t = w1_ref[:, t * st:(t + 1) * st].astype(BF16)
        h1 = jnp.dot(h2v, w1t, preferred_element_type=F32) \
            + b1_ref[:, t * st:(t + 1) * st]
        g = (0.5 * h1 * (1.0 + jax.lax.erf(h1 * (2.0 ** -0.5)))).astype(BF16)
        w2t = w2_ref[t * st:(t + 1) * st, :].astype(BF16)
        d = jnp.dot(g, w2t, preferred_element_type=F32)
        acc = d if acc is None else acc + d
    out_ref[...] += acc


def kernel(x, y, gamma1, beta1, Wg, bg, Wqkv, Wp, bp, gamma2, beta2,
           W1, b1, W2, b2):
    g1 = gamma1.reshape(1, DIM)
    be1 = beta1.reshape(1, DIM)
    g2 = gamma2.reshape(1, DIM)
    be2 = beta2.reshape(1, DIM)
    bgr = bg.reshape(1, E)
    bpr = bp.reshape(1, DIM)
    b1r = b1.reshape(1, 4 * DIM)
    b2r = b2.reshape(1, DIM)

    yn_b, x_b, w = pl.pallas_call(
        _pre_kernel,
        grid=(RBM,),
        in_specs=[
            pl.BlockSpec((BTM, DIM), lambda r: (r, 0)),
            pl.BlockSpec((BTM, DIM), lambda r: (r, 0)),
            pl.BlockSpec((1, DIM), lambda r: (0, 0)),
            pl.BlockSpec((1, DIM), lambda r: (0, 0)),
            pl.BlockSpec((DIM, E), lambda r: (0, 0)),
            pl.BlockSpec((1, E), lambda r: (0, 0)),
        ],
        out_specs=[
            pl.BlockSpec((BTM, DIM), lambda r: (r, 0)),
            pl.BlockSpec((BTM, DIM), lambda r: (r, 0)),
            pl.BlockSpec((BTM, E), lambda r: (r, 0)),
        ],
        out_shape=[
            jax.ShapeDtypeStruct((B, DIM), BF16),
            jax.ShapeDtypeStruct((B, DIM), BF16),
            jax.ShapeDtypeStruct((B, E), F32),
        ],
        compiler_params=pltpu.CompilerParams(
            dimension_semantics=("parallel",)),
    )(x, y, g1, be1, Wg, bgr)

    mask = _sc_top2_mask(w)

    wq_b, wkv_b, Wp_b = pl.pallas_call(
        _prep_kernel,
        grid=(NQ, E),
        in_specs=[
            pl.BlockSpec((1, DIM, QW), lambda n, e: (e, 0, n)),
            pl.BlockSpec((1, DIM, QW), lambda n, e: (e, 0, NQ + n)),
            pl.BlockSpec((1, DIM, QW), lambda n, e: (e, 0, 2 * NQ + n)),
            pl.BlockSpec((DIM, QW), lambda n, e: (0, n)),
        ],
        out_specs=[
            pl.BlockSpec((1, DIM, QW), lambda n, e: (e, 0, n)),
            pl.BlockSpec((1, DIM, QW), lambda n, e: (e, 0, n)),
            pl.BlockSpec((DIM, QW), lambda n, e: (0, n)),
        ],
        out_shape=[
            jax.ShapeDtypeStruct((E, DIM, DIM), BF16),
            jax.ShapeDtypeStruct((E, DIM, DIM), BF16),
            jax.ShapeDtypeStruct((DIM, DIM), BF16),
        ],
        compiler_params=pltpu.CompilerParams(
            dimension_semantics=("parallel", "arbitrary")),
    )(Wqkv, Wqkv, Wqkv, Wp)

    attn = pl.pallas_call(
        _qkv_kernel,
        grid=(RB, E),
        in_specs=[
            pl.BlockSpec((BT, DIM), lambda r, e: (r, 0)),
            pl.BlockSpec((BT, DIM), lambda r, e: (r, 0)),
            pl.BlockSpec((1, DIM, DIM), lambda r, e: (e, 0, 0)),
            pl.BlockSpec((1, DIM, DIM), lambda r, e: (e, 0, 0)),
            pl.BlockSpec((BT, E), lambda r, e: (r, 0)),
            pl.BlockSpec((BT, E), lambda r, e: (r, 0)),
        ],
        out_specs=pl.BlockSpec((BT, DIM), lambda r, e: (r, 0)),
        out_shape=jax.ShapeDtypeStruct((B, DIM), BF16),
        scratch_shapes=[pltpu.VMEM((BT, DIM), F32)],
        compiler_params=pltpu.CompilerParams(
            dimension_semantics=("parallel", "arbitrary")),
    )(x_b, yn_b, wq_b, wkv_b, w, mask)

    out1, h2 = pl.pallas_call(
        _proj_kernel,
        grid=(RBM,),
        in_specs=[
            pl.BlockSpec((BTM, DIM), lambda r: (r, 0)),
            pl.BlockSpec((BTM, DIM), lambda r: (r, 0)),
            pl.BlockSpec((DIM, DIM), lambda r: (0, 0)),
            pl.BlockSpec((1, DIM), lambda r: (0, 0)),
            pl.BlockSpec((1, DIM), lambda r: (0, 0)),
            pl.BlockSpec((1, DIM), lambda r: (0, 0)),
            pl.BlockSpec((1, DIM), lambda r: (0, 0)),
        ],
        out_specs=[
            pl.BlockSpec((BTM, DIM), lambda r: (r, 0)),
            pl.BlockSpec((BTM, DIM), lambda r: (r, 0)),
        ],
        out_shape=[
            jax.ShapeDtypeStruct((B, DIM), BF16),
            jax.ShapeDtypeStruct((B, DIM), BF16),
        ],
        compiler_params=pltpu.CompilerParams(
            dimension_semantics=("parallel",)),
    )(y, attn, Wp_b, bpr, g2, be2, b2r)

    out = pl.pallas_call(
        _mlp_kernel,
        grid=(RBM, N1),
        in_specs=[
            pl.BlockSpec((BTM, DIM), lambda r, n: (r, 0)),
            pl.BlockSpec((BTM, DIM), lambda r, n: (r, 0)),
            pl.BlockSpec((DIM, HB), lambda r, n: (0, n)),
            pl.BlockSpec((1, HB), lambda r, n: (0, n)),
            pl.BlockSpec((HB, DIM), lambda r, n: (n, 0)),
        ],
        out_specs=pl.BlockSpec((BTM, DIM), lambda r, n: (r, 0)),
        out_shape=jax.ShapeDtypeStruct((B, DIM), F32),
        compiler_params=pltpu.CompilerParams(
            dimension_semantics=("parallel", "arbitrary")),
    )(out1, h2, W1, b1r, W2)

    return out


# speedup vs baseline: 1.0054x; 1.0054x over previous
import jax
import jax.numpy as jnp
from jax.experimental import pallas as pl
from jax.experimental.pallas import tpu as pltpu
from jax.experimental.pallas import tpu_sc as plsc

DIM = 2048
B = 2048
E = 4
H = 4
DH = DIM // H
BT = 512
RB = B // BT
NQ = 4
QW = DIM // NQ
N1 = 16
HB = (4 * DIM) // N1
BTM = 1024
RBM = B // BTM

F32 = jnp.float32
BF16 = jnp.bfloat16


def _prep_kernel(q_ref, k_ref, v_ref, wp_ref, wq_ref, wkv_ref, wpb_ref):
    e = pl.program_id(1)
    wq_ref[...] = q_ref[...].astype(BF16)
    wkv_ref[...] = (k_ref[...] + v_ref[...]).astype(BF16)

    @pl.when(e == 0)
    def _():
        wp = wp_ref[...].reshape(DH, H, QW)
        wpb_ref[...] = (
            wp.transpose(1, 0, 2).reshape(DIM, QW).astype(BF16))


def _sc_top2_mask(p):
    mesh = plsc.ScalarSubcoreMesh(axis_name="core", num_cores=2)
    half = B // 2
    hw = half * E

    @pl.kernel(out_type=jax.ShapeDtypeStruct((B * E,), F32), mesh=mesh,
               scratch_types=[pltpu.SMEM((hw,), F32),
                              pltpu.SMEM((hw,), F32),
                              pltpu.SemaphoreType.DMA])
    def sc_mask(p_hbm, m_hbm, pin, mout, sem):
        c = jax.lax.axis_index("core")
        pltpu.async_copy(p_hbm.at[pl.ds(c * hw, hw)], pin, sem).wait()

        @pl.loop(0, half)
        def _(t):
            p0 = pin[4 * t + 0]
            p1 = pin[4 * t + 1]
            p2 = pin[4 * t + 2]
            p3 = pin[4 * t + 3]
            b01 = p1 > p0
            m01 = jnp.where(b01, p1, p0)
            i01 = jnp.where(b01, 1, 0)
            b23 = p3 > p2
            m23 = jnp.where(b23, p3, p2)
            i23 = jnp.where(b23, 3, 2)
            i1 = jnp.where(m23 > m01, i23, i01)
            q0 = jnp.where(i1 == 0, -1.0, p0)
            q1 = jnp.where(i1 == 1, -1.0, p1)
            q2 = jnp.where(i1 == 2, -1.0, p2)
            q3 = jnp.where(i1 == 3, -1.0, p3)
            c01 = q1 > q0
            n01 = jnp.where(c01, q1, q0)
            j01 = jnp.where(c01, 1, 0)
            c23 = q3 > q2
            n23 = jnp.where(c23, q3, q2)
            j23 = jnp.where(c23, 3, 2)
            i2 = jnp.where(n23 > n01, j23, j01)
            mout[4 * t + 0] = jnp.where((i1 == 0) | (i2 == 0), 1.0, 0.0)
            mout[4 * t + 1] = jnp.where((i1 == 1) | (i2 == 1), 1.0, 0.0)
            mout[4 * t + 2] = jnp.where((i1 == 2) | (i2 == 2), 1.0, 0.0)
            mout[4 * t + 3] = jnp.where((i1 == 3) | (i2 == 3), 1.0, 0.0)

        pltpu.async_copy(mout, m_hbm.at[pl.ds(c * hw, hw)], sem).wait()

    return sc_mask(p.reshape(B * E)).reshape(B, E)


def _pre_kernel(x_ref, y_ref, g1_ref, b1_ref, wg_ref, bg_ref,
                yn_ref, xb_ref, w_ref):
    xv = x_ref[...]
    yv = y_ref[...]
    m = jnp.mean(yv, axis=1, keepdims=True)
    v = jnp.mean((yv - m) ** 2, axis=1, keepdims=True)
    yn = (yv - m) / jnp.sqrt(v + 1e-5) * g1_ref[...] + b1_ref[...]
    yn_ref[...] = yn.astype(BF16)
    xb_ref[...] = xv.astype(BF16)
    logits = jnp.dot(xv.astype(BF16), wg_ref[...].astype(BF16),
                     preferred_element_type=F32) + bg_ref[...]
    m1 = jnp.max(logits, axis=1, keepdims=True)
    p = jnp.exp(logits - m1)
    w_ref[...] = p / jnp.sum(p, axis=1, keepdims=True)


def _qkv_kernel(xb_ref, yn_ref, wq_ref, wkv_ref, w_ref, mask_ref,
                attn_ref, acc_ref):
    e = pl.program_id(1)
    contrib = (jnp.dot(yn_ref[...], wq_ref[0], preferred_element_type=F32)
               + jnp.dot(xb_ref[...], wkv_ref[0], preferred_element_type=F32))
    lane = jax.lax.broadcasted_iota(jnp.int32, (BT, E), 1)
    wcol = jnp.sum(jnp.where(lane == e, w_ref[...] * mask_ref[...], 0.0),
                   axis=1, keepdims=True)
    contrib = contrib * wcol

    @pl.when(e == 0)
    def _():
        acc_ref[...] = contrib

    @pl.when(e > 0)
    def _():
        acc_ref[...] += contrib

    @pl.when(e == E - 1)
    def _():
        acc = acc_ref[...]
        scale = DH ** -0.5
        heads = [acc[:, h * DH:(h + 1) * DH] for h in range(H)]
        s = [[None] * H for _ in range(H)]
        for h in range(H):
            for g in range(h, H):
                s[h][g] = jnp.sum(heads[h] * heads[g], axis=1,
                                  keepdims=True) * scale
                s[g][h] = s[h][g]
        outs = []
        for h in range(H):
            mx = s[h][0]
            for g in range(1, H):
                mx = jnp.maximum(mx, s[h][g])
            es = [jnp.exp(s[h][g] - mx) for g in range(H)]
            den = es[0] + es[1] + es[2] + es[3]
            o = (es[0] / den) * heads[0]
            for g in range(1, H):
                o += (es[g] / den) * heads[g]
            outs.append(o)
        attn_ref[...] = jnp.concatenate(outs, axis=1).astype(BF16)


def _proj_kernel(y_ref, attn_ref, wp_ref, bp_ref, g2_ref, be2_ref, b2_ref,
                 out1_ref, h2_ref):
    yv = y_ref[...]
    m = jnp.mean(yv, axis=1, keepdims=True)
    v = jnp.mean((yv - m) ** 2, axis=1, keepdims=True)
    h2 = (yv - m) / jnp.sqrt(v + 1e-5) * g2_ref[...] + be2_ref[...]
    h2_ref[...] = h2.astype(BF16)
    out1_ref[...] = (yv
                     + jnp.dot(attn_ref[...], wp_ref[...],
                               preferred_element_type=F32)
                     + bp_ref[...] + b2_ref[...]).astype(BF16)


def _mlp_kernel(out1_ref, h2_ref, w1_ref, b1_ref, w2_ref, out_ref):
    n = pl.program_id(1)

    @pl.when(n == 0)
    def _():
        out_ref[...] = out1_ref[...].astype(F32)

    h2v = h2_ref[...]
    st = HB // 2
    acc = None
    for t in range(2):
        w1t = w1_ref[:, t * st:(t + 1) * st].astype(BF16)
        h1 = jnp.dot(h2v, w1t, preferred_element_type=F32) \
            + b1_ref[:, t * st:(t + 1) * st]
        g = (0.5 * h1 * (1.0 + jax.lax.erf(h1 * (2.0 ** -0.5)))).astype(BF16)
        w2t = w2_ref[t * st:(t + 1) * st, :].astype(BF16)
        d = jnp.dot(g, w2t, preferred_element_type=F32)
        acc = d if acc is None else acc + d
    out_ref[...] += acc


def kernel(x, y, gamma1, beta1, Wg, bg, Wqkv, Wp, bp, gamma2, beta2,
           W1, b1, W2, b2):
    g1 = gamma1.reshape(1, DIM)
    be1 = beta1.reshape(1, DIM)
    g2 = gamma2.reshape(1, DIM)
    be2 = beta2.reshape(1, DIM)
    bgr = bg.reshape(1, E)
    bpr = bp.reshape(1, DIM)
    b1r = b1.reshape(1, 4 * DIM)
    b2r = b2.reshape(1, DIM)

    yn_b, x_b, w = pl.pallas_call(
        _pre_kernel,
        grid=(RB,),
        in_specs=[
            pl.BlockSpec((BT, DIM), lambda r: (r, 0)),
            pl.BlockSpec((BT, DIM), lambda r: (r, 0)),
            pl.BlockSpec((1, DIM), lambda r: (0, 0)),
            pl.BlockSpec((1, DIM), lambda r: (0, 0)),
            pl.BlockSpec((DIM, E), lambda r: (0, 0)),
            pl.BlockSpec((1, E), lambda r: (0, 0)),
        ],
        out_specs=[
            pl.BlockSpec((BT, DIM), lambda r: (r, 0)),
            pl.BlockSpec((BT, DIM), lambda r: (r, 0)),
            pl.BlockSpec((BT, E), lambda r: (r, 0)),
        ],
        out_shape=[
            jax.ShapeDtypeStruct((B, DIM), BF16),
            jax.ShapeDtypeStruct((B, DIM), BF16),
            jax.ShapeDtypeStruct((B, E), F32),
        ],
        compiler_params=pltpu.CompilerParams(
            dimension_semantics=("parallel",)),
    )(x, y, g1, be1, Wg, bgr)

    mask = _sc_top2_mask(w)

    wq_b, wkv_b, Wp_b = pl.pallas_call(
        _prep_kernel,
        grid=(NQ, E),
        in_specs=[
            pl.BlockSpec((1, DIM, QW), lambda n, e: (e, 0, n)),
            pl.BlockSpec((1, DIM, QW), lambda n, e: (e, 0, NQ + n)),
            pl.BlockSpec((1, DIM, QW), lambda n, e: (e, 0, 2 * NQ + n)),
            pl.BlockSpec((DIM, QW), lambda n, e: (0, n)),
        ],
        out_specs=[
            pl.BlockSpec((1, DIM, QW), lambda n, e: (e, 0, n)),
            pl.BlockSpec((1, DIM, QW), lambda n, e: (e, 0, n)),
            pl.BlockSpec((DIM, QW), lambda n, e: (0, n)),
        ],
        out_shape=[
            jax.ShapeDtypeStruct((E, DIM, DIM), BF16),
            jax.ShapeDtypeStruct((E, DIM, DIM), BF16),
            jax.ShapeDtypeStruct((DIM, DIM), BF16),
        ],
        compiler_params=pltpu.CompilerParams(
            dimension_semantics=("parallel", "arbitrary")),
    )(Wqkv, Wqkv, Wqkv, Wp)

    attn = pl.pallas_call(
        _qkv_kernel,
        grid=(RB, E),
        in_specs=[
            pl.BlockSpec((BT, DIM), lambda r, e: (r, 0)),
            pl.BlockSpec((BT, DIM), lambda r, e: (r, 0)),
            pl.BlockSpec((1, DIM, DIM), lambda r, e: (e, 0, 0)),
            pl.BlockSpec((1, DIM, DIM), lambda r, e: (e, 0, 0)),
            pl.BlockSpec((BT, E), lambda r, e: (r, 0)),
            pl.BlockSpec((BT, E), lambda r, e: (r, 0)),
        ],
        out_specs=pl.BlockSpec((BT, DIM), lambda r, e: (r, 0)),
        out_shape=jax.ShapeDtypeStruct((B, DIM), BF16),
        scratch_shapes=[pltpu.VMEM((BT, DIM), F32)],
        compiler_params=pltpu.CompilerParams(
            dimension_semantics=("parallel", "arbitrary")),
    )(x_b, yn_b, wq_b, wkv_b, w, mask)

    out1, h2 = pl.pallas_call(
        _proj_kernel,
        grid=(RB,),
        in_specs=[
            pl.BlockSpec((BT, DIM), lambda r: (r, 0)),
            pl.BlockSpec((BT, DIM), lambda r: (r, 0)),
            pl.BlockSpec((DIM, DIM), lambda r: (0, 0)),
            pl.BlockSpec((1, DIM), lambda r: (0, 0)),
            pl.BlockSpec((1, DIM), lambda r: (0, 0)),
            pl.BlockSpec((1, DIM), lambda r: (0, 0)),
            pl.BlockSpec((1, DIM), lambda r: (0, 0)),
        ],
        out_specs=[
            pl.BlockSpec((BT, DIM), lambda r: (r, 0)),
            pl.BlockSpec((BT, DIM), lambda r: (r, 0)),
        ],
        out_shape=[
            jax.ShapeDtypeStruct((B, DIM), BF16),
            jax.ShapeDtypeStruct((B, DIM), BF16),
        ],
        compiler_params=pltpu.CompilerParams(
            dimension_semantics=("parallel",)),
    )(y, attn, Wp_b, bpr, g2, be2, b2r)

    out = pl.pallas_call(
        _mlp_kernel,
        grid=(RBM, N1),
        in_specs=[
            pl.BlockSpec((BTM, DIM), lambda r, n: (r, 0)),
            pl.BlockSpec((BTM, DIM), lambda r, n: (r, 0)),
            pl.BlockSpec((DIM, HB), lambda r, n: (0, n)),
            pl.BlockSpec((1, HB), lambda r, n: (0, n)),
            pl.BlockSpec((HB, DIM), lambda r, n: (n, 0)),
        ],
        out_specs=pl.BlockSpec((BTM, DIM), lambda r, n: (r, 0)),
        out_shape=jax.ShapeDtypeStruct((B, DIM), F32),
        compiler_params=pltpu.CompilerParams(
            dimension_semantics=("parallel", "arbitrary")),
    )(out1, h2, W1, b1r, W2)

    return out
